# Initial kernel scaffold; baseline (speedup 1.0000x reference)
import jax
import jax.numpy as jnp
from jax.experimental import pallas as pl


def kernel(edge_index, edge_weight, params):
    raise NotImplementedError("write your pallas kernel here")


# speedup vs baseline: 4.4866x; 4.4866x over previous
import functools

import jax
import jax.numpy as jnp
from jax import lax
from jax.experimental import pallas as pl
from jax.experimental.pallas import tpu as pltpu
from jax.experimental.pallas import tpu_sc as plsc

_NC = 2
_NS = 16
_LANES = 16


def _rawproj_combine(raw, Wp, bp, learned, alpha11):
    N, RAW = raw.shape
    D = Wp.shape[1]
    BM = 512

    def body(a_ref, raw_ref, w_ref, b_ref, learned_ref, out_ref):
        a = a_ref[0, 0]
        acc = jnp.dot(raw_ref[...], w_ref[...], preferred_element_type=jnp.float32)
        out_ref[...] = a * (acc + b_ref[...]) + (1.0 - a) * learned_ref[...]

    return pl.pallas_call(
        body,
        grid=(N // BM,),
        in_specs=[
            pl.BlockSpec(memory_space=pltpu.SMEM),
            pl.BlockSpec((BM, RAW), lambda i: (i, 0)),
            pl.BlockSpec((RAW, D), lambda i: (0, 0)),
            pl.BlockSpec((1, D), lambda i: (0, 0)),
            pl.BlockSpec((BM, D), lambda i: (i, 0)),
        ],
        out_specs=pl.BlockSpec((BM, D), lambda i: (i, 0)),
        out_shape=jax.ShapeDtypeStruct((N, D), jnp.float32),
    )(alpha11, raw, Wp, bp.reshape(1, D), learned)


def _dinv_kernel(S):
    I = S.shape[0]
    BM = 512

    def body(s_ref, out_ref):
        out_ref[...] = lax.rsqrt(jnp.sum(s_ref[...], axis=1, keepdims=True) + 1e-7)

    return pl.pallas_call(
        body,
        grid=(I // BM,),
        in_specs=[pl.BlockSpec((BM, I), lambda i: (i, 0))],
        out_specs=pl.BlockSpec((BM, 1), lambda i: (i, 0)),
        out_shape=jax.ShapeDtypeStruct((I, 1), jnp.float32),
    )(S)


def _simprop(S, dinv, x, alpha11):
    I = S.shape[0]
    D = x.shape[1]
    BM = 512
    BK = 512
    nk = I // BK

    def body(a_ref, s_ref, x_ref, dk_ref, dm_ref, out_ref):
        k = pl.program_id(1)

        @pl.when(k == 0)
        def _init():
            out_ref[...] = jnp.zeros_like(out_ref)

        xs = dk_ref[...] * x_ref[...]
        out_ref[...] += jnp.dot(s_ref[...], xs, preferred_element_type=jnp.float32)

        @pl.when(k == nk - 1)
        def _fini():
            out_ref[...] = a_ref[0, 0] * dm_ref[...] * out_ref[...]

    return pl.pallas_call(
        body,
        grid=(I // BM, nk),
        in_specs=[
            pl.BlockSpec(memory_space=pltpu.SMEM),
            pl.BlockSpec((BM, BK), lambda i, k: (i, k)),
            pl.BlockSpec((BK, D), lambda i, k: (k, 0)),
            pl.BlockSpec((BK, 1), lambda i, k: (k, 0)),
            pl.BlockSpec((BM, 1), lambda i, k: (i, 0)),
        ],
        out_specs=pl.BlockSpec((BM, D), lambda i, k: (i, 0)),
        out_shape=jax.ShapeDtypeStruct((I, D), jnp.float32),
        compiler_params=pltpu.CompilerParams(
            dimension_semantics=("parallel", "arbitrary")),
    )(alpha11, S, x, dinv, dinv)


def _sc_scatter(rows, cols, w, embeds, zeros_slab):
    E = rows.shape[0]
    N, D = embeds.shape
    NW = _NC * _NS
    e_per_w = E // NW
    CH = 128
    n_chunks = e_per_w // CH
    rps = N // _NS

    mesh = plsc.VectorSubcoreMesh(core_axis_name="c", subcore_axis_name="s")

    @functools.partial(
        pl.kernel,
        mesh=mesh,
        compiler_params=pltpu.CompilerParams(use_tc_tiling_on_sc=False),
        out_type=jax.ShapeDtypeStruct((_NC * N, D), jnp.float32),
        scratch_types=[
            pltpu.VMEM((CH,), jnp.int32),
            pltpu.VMEM((CH,), jnp.int32),
            pltpu.VMEM((CH,), jnp.float32),
            pltpu.VMEM((CH, D), jnp.float32),
            pltpu.VMEM_SHARED((N, D), jnp.float32),
            pltpu.SemaphoreType.DMA,
        ],
    )
    def k(rows_hbm, cols_hbm, w_hbm, emb_hbm, zero_hbm, out_hbm,
          cols_v, rowsi_v, w_v, data_v, acc_sh, sem):
        cid = lax.axis_index("c")
        sid = lax.axis_index("s")
        wid = sid * _NC + cid

        pltpu.sync_copy(zero_hbm, acc_sh.at[pl.ds(sid * rps, rps)])
        plsc.subcore_barrier()

        base = wid * e_per_w

        def chunk_body(i, carry):
            off = base + i * CH
            pltpu.sync_copy(cols_hbm.at[pl.ds(off, CH)], cols_v)
            pltpu.sync_copy(rows_hbm.at[pl.ds(off, CH)], rowsi_v)
            pltpu.sync_copy(w_hbm.at[pl.ds(off, CH)], w_v)
            pltpu.async_copy(emb_hbm.at[cols_v], data_v, sem).wait()

            def scale_group(g, c):
                w16 = w_v[pl.ds(g * _LANES, _LANES)]
                for el in range(_LANES):
                    wv = jnp.full((_LANES,), w16[el], dtype=jnp.float32)
                    e = g * _LANES + el
                    for j in range(D // _LANES):
                        sl = pl.ds(j * _LANES, _LANES)
                        data_v[e, sl] = data_v[e, sl] * wv
                return c

            lax.fori_loop(0, CH // _LANES, scale_group, 0)
            pltpu.sync_copy(data_v, acc_sh.at[rowsi_v], add=True)
            return carry

        lax.fori_loop(0, n_chunks, chunk_body, 0)
        plsc.subcore_barrier()
        pltpu.sync_copy(acc_sh.at[pl.ds(sid * rps, rps)],
                        out_hbm.at[pl.ds(cid * N + sid * rps, rps)])

    return k(rows, cols, w, embeds, zeros_slab)


def _pe_combine(p0, p1, simy, pos, W, b, n_user_blocks):
    N, D = p0.shape
    BM = 512

    def body(p0_ref, p1_ref, sim_ref, pos_ref, w_ref, b_ref, out_ref):
        i = pl.program_id(0)
        cur = p0_ref[...] + p1_ref[...]
        cur = jnp.where(i >= n_user_blocks, cur + sim_ref[...], cur)
        gate = jax.nn.sigmoid(
            jnp.dot(cur, w_ref[...], preferred_element_type=jnp.float32)
            + b_ref[...])
        out_ref[...] = cur + gate * pos_ref[...]

    return pl.pallas_call(
        body,
        grid=(N // BM,),
        in_specs=[
            pl.BlockSpec((BM, D), lambda i: (i, 0)),
            pl.BlockSpec((BM, D), lambda i: (i, 0)),
            pl.BlockSpec((BM, D), lambda i: (jnp.maximum(i - n_user_blocks, 0), 0)),
            pl.BlockSpec((BM, D), lambda i: (i, 0)),
            pl.BlockSpec((D, D), lambda i: (0, 0)),
            pl.BlockSpec((1, D), lambda i: (0, 0)),
        ],
        out_specs=pl.BlockSpec((BM, D), lambda i: (i, 0)),
        out_shape=jax.ShapeDtypeStruct((N, D), jnp.float32),
    )(p0, p1, simy, pos, W, b.reshape(1, D))


def _transformer(x, p, H):
    L, D = x.shape
    F = p['W1'].shape[1]
    dh = D // H
    BQ = 512
    scale = 1.0 / (dh ** 0.5)

    def ln(v, g, b):
        m = jnp.mean(v, axis=-1, keepdims=True)
        var = jnp.mean((v - m) * (v - m), axis=-1, keepdims=True)
        return (v - m) / jnp.sqrt(var + 1e-5) * g + b

    def body(x_ref, xb_ref, wq_ref, bq_ref, wk_ref, bk_ref, wv_ref, bv_ref,
             wo_ref, bo_ref, g1_ref, be1_ref, g2_ref, be2_ref,
             w1_ref, b1_ref, w2_ref, b2_ref, out_ref):
        xf = x_ref[...]
        xb = xb_ref[...]
        q = jnp.dot(xb, wq_ref[...], preferred_element_type=jnp.float32) + bq_ref[...]
        kk = jnp.dot(xf, wk_ref[...], preferred_element_type=jnp.float32) + bk_ref[...]
        vv = jnp.dot(xf, wv_ref[...], preferred_element_type=jnp.float32) + bv_ref[...]
        outs = []
        for h in range(H):
            sl = slice(h * dh, (h + 1) * dh)
            qh = q[:, sl]
            kh = kk[:, sl]
            vh = vv[:, sl]
            s = lax.dot_general(qh, kh, (((1,), (1,)), ((), ())),
                                preferred_element_type=jnp.float32) * scale
            m = jnp.max(s, axis=-1, keepdims=True)
            e = jnp.exp(s - m)
            denom = jnp.sum(e, axis=-1, keepdims=True)
            outs.append(
                jnp.dot(e, vh, preferred_element_type=jnp.float32) / denom)
        o = jnp.concatenate(outs, axis=-1)
        h1 = xb + jnp.dot(o, wo_ref[...], preferred_element_type=jnp.float32) + bo_ref[...]
        h1 = ln(h1, g1_ref[...], be1_ref[...])
        ff = jnp.maximum(
            jnp.dot(h1, w1_ref[...], preferred_element_type=jnp.float32) + b1_ref[...],
            0.0)
        ff = jnp.dot(ff, w2_ref[...], preferred_element_type=jnp.float32) + b2_ref[...]
        out_ref[...] = ln(h1 + ff, g2_ref[...], be2_ref[...])

    full = lambda shape: pl.BlockSpec(shape, lambda i: (0, 0))
    return pl.pallas_call(
        body,
        grid=(L // BQ,),
        in_specs=[
            pl.BlockSpec((L, D), lambda i: (0, 0)),
            pl.BlockSpec((BQ, D), lambda i: (i, 0)),
            full((D, D)), full((1, D)),
            full((D, D)), full((1, D)),
            full((D, D)), full((1, D)),
            full((D, D)), full((1, D)),
            full((1, D)), full((1, D)),
            full((1, D)), full((1, D)),
            full((D, F)), full((1, F)),
            full((F, D)), full((1, D)),
        ],
        out_specs=pl.BlockSpec((BQ, D), lambda i: (i, 0)),
        out_shape=jax.ShapeDtypeStruct((L, D), jnp.float32),
    )(x, x,
      p['Wq'], p['bq'].reshape(1, D),
      p['Wk'], p['bk'].reshape(1, D),
      p['Wv'], p['bv'].reshape(1, D),
      p['Wo'], p['bo'].reshape(1, D),
      p['ln1_g'].reshape(1, D), p['ln1_b'].reshape(1, D),
      p['ln2_g'].reshape(1, D), p['ln2_b'].reshape(1, D),
      p['W1'], p['b1'].reshape(1, F),
      p['W2'], p['b2'].reshape(1, D))


def _wsum(e0, e1, e2):
    N, D = e0.shape
    BM = 512

    def body(a_ref, b_ref, c_ref, out_ref):
        out_ref[...] = a_ref[...] + 0.75 * b_ref[...] + 0.5 * c_ref[...]

    spec = pl.BlockSpec((BM, D), lambda i: (i, 0))
    return pl.pallas_call(
        body,
        grid=(N // BM,),
        in_specs=[spec, spec, spec],
        out_specs=spec,
        out_shape=jax.ShapeDtypeStruct((N, D), jnp.float32),
    )(e0, e1, e2)


def kernel(edge_index, edge_weight, params):
    p = params
    U = p['user_emb'].shape[0]
    N, RAW = p['raw_emb'].shape
    D = p['W_proj'].shape[1]
    H = 2

    rows = edge_index[0]
    cols = edge_index[1]
    ew = edge_weight.astype(jnp.float32)
    alpha11 = jnp.reshape(p['alpha'], (1, 1)).astype(jnp.float32)
    learned = jnp.concatenate([p['user_emb'], p['item_emb']], axis=0)
    zeros_slab = jnp.zeros((N // _NS, D), jnp.float32)

    combined = _rawproj_combine(p['raw_emb'], p['W_proj'], p['b_proj'],
                                learned, alpha11)
    dinv = _dinv_kernel(p['visual_sim'])

    n_user_blocks = U // 512
    cur = combined
    stages = [combined]
    for _ in range(2):
        simy = _simprop(p['visual_sim'], dinv, cur[U:], alpha11)
        parts = _sc_scatter(rows, cols, ew, cur, zeros_slab)
        cur2 = _pe_combine(parts[:N], parts[N:], simy, p['pos_table'],
                           p['pe_gate_W'], p['pe_gate_b'], n_user_blocks)
        u = _transformer(cur2[:U], p['user_enc'], H)
        it = _transformer(cur2[U:], p['item_enc'], H)
        cur = jnp.concatenate([u, it], axis=0)
        stages.append(cur)

    final = _wsum(stages[0], stages[1], stages[2])
    return final, final[:U], final[U:]


# speedup vs baseline: 5.4921x; 1.2241x over previous
import functools

import jax
import jax.numpy as jnp
from jax import lax
from jax.experimental import pallas as pl
from jax.experimental.pallas import tpu as pltpu
from jax.experimental.pallas import tpu_sc as plsc

_NC = 2
_NS = 16
_LANES = 16


def _rawproj_combine(raw, Wp, bp, learned, alpha11):
    N, RAW = raw.shape
    D = Wp.shape[1]
    BM = 512

    def body(a_ref, raw_ref, w_ref, b_ref, learned_ref, out_ref):
        a = a_ref[0, 0]
        acc = jnp.dot(raw_ref[...], w_ref[...], preferred_element_type=jnp.float32)
        out_ref[...] = a * (acc + b_ref[...]) + (1.0 - a) * learned_ref[...]

    return pl.pallas_call(
        body,
        grid=(N // BM,),
        in_specs=[
            pl.BlockSpec(memory_space=pltpu.SMEM),
            pl.BlockSpec((BM, RAW), lambda i: (i, 0)),
            pl.BlockSpec((RAW, D), lambda i: (0, 0)),
            pl.BlockSpec((1, D), lambda i: (0, 0)),
            pl.BlockSpec((BM, D), lambda i: (i, 0)),
        ],
        out_specs=pl.BlockSpec((BM, D), lambda i: (i, 0)),
        out_shape=jax.ShapeDtypeStruct((N, D), jnp.float32),
    )(alpha11, raw, Wp, bp.reshape(1, D), learned)


def _dinv_kernel(S):
    I = S.shape[0]
    BM = 512

    def body(s_ref, out_ref):
        out_ref[...] = lax.rsqrt(jnp.sum(s_ref[...], axis=1, keepdims=True) + 1e-7)

    return pl.pallas_call(
        body,
        grid=(I // BM,),
        in_specs=[pl.BlockSpec((BM, I), lambda i: (i, 0))],
        out_specs=pl.BlockSpec((BM, 1), lambda i: (i, 0)),
        out_shape=jax.ShapeDtypeStruct((I, 1), jnp.float32),
    )(S)


def _simprop(S, dinv, x, alpha11):
    I = S.shape[0]
    D = x.shape[1]
    BM = 512
    BK = 512
    nk = I // BK

    def body(a_ref, s_ref, x_ref, dk_ref, dm_ref, out_ref):
        k = pl.program_id(1)

        @pl.when(k == 0)
        def _init():
            out_ref[...] = jnp.zeros_like(out_ref)

        xs = dk_ref[...] * x_ref[...]
        out_ref[...] += jnp.dot(s_ref[...], xs, preferred_element_type=jnp.float32)

        @pl.when(k == nk - 1)
        def _fini():
            out_ref[...] = a_ref[0, 0] * dm_ref[...] * out_ref[...]

    return pl.pallas_call(
        body,
        grid=(I // BM, nk),
        in_specs=[
            pl.BlockSpec(memory_space=pltpu.SMEM),
            pl.BlockSpec((BM, BK), lambda i, k: (i, k)),
            pl.BlockSpec((BK, D), lambda i, k: (k, 0)),
            pl.BlockSpec((BK, 1), lambda i, k: (k, 0)),
            pl.BlockSpec((BM, 1), lambda i, k: (i, 0)),
        ],
        out_specs=pl.BlockSpec((BM, D), lambda i, k: (i, 0)),
        out_shape=jax.ShapeDtypeStruct((I, D), jnp.float32),
        compiler_params=pltpu.CompilerParams(
            dimension_semantics=("parallel", "arbitrary")),
    )(alpha11, S, x, dinv, dinv)


_CH = 128


def _sc_scatter(packed, wchunk, embeds, zeros_slab):
    total_chunks, _, CH = packed.shape
    N, D = embeds.shape
    NW = _NC * _NS
    n_chunks = total_chunks // NW
    rps = N // _NS

    mesh = plsc.VectorSubcoreMesh(core_axis_name="c", subcore_axis_name="s")

    @functools.partial(
        pl.kernel,
        mesh=mesh,
        compiler_params=pltpu.CompilerParams(use_tc_tiling_on_sc=False),
        out_type=jax.ShapeDtypeStruct((_NC * N, D), jnp.float32),
        scratch_types=[
            pltpu.VMEM((2, CH), jnp.int32),
            pltpu.VMEM((2, CH), jnp.int32),
            pltpu.VMEM((CH,), jnp.float32),
            pltpu.VMEM((CH,), jnp.float32),
            pltpu.VMEM((CH, D), jnp.float32),
            pltpu.VMEM((CH, D), jnp.float32),
            pltpu.VMEM_SHARED((N, D), jnp.float32),
            pltpu.SemaphoreType.DMA,
            pltpu.SemaphoreType.DMA,
        ],
    )
    def k(pk_hbm, wc_hbm, emb_hbm, zero_hbm, out_hbm,
          pkt0, pkt1, wv0, wv1, dat0, dat1, acc_sh, sem0, sem1):
        cid = lax.axis_index("c")
        sid = lax.axis_index("s")
        wid = sid * _NC + cid
        base = wid * n_chunks

        pltpu.sync_copy(zero_hbm, acc_sh.at[pl.ds(sid * rps, rps)])
        pltpu.sync_copy(pk_hbm.at[base], pkt0)
        pltpu.sync_copy(wc_hbm.at[pl.ds(base * CH, CH)], wv0)
        pltpu.async_copy(emb_hbm.at[pkt0.at[1]], dat0, sem0)
        plsc.subcore_barrier()

        bufs = ((pkt0, wv0, dat0, sem0), (pkt1, wv1, dat1, sem1))

        def pair(g, c):
            for b in range(2):
                i = 2 * g + b
                pkt, wv_, dat, sem = bufs[b]
                npkt, nwv, ndat, nsem = bufs[1 - b]

                @pl.when(i + 1 < n_chunks)
                def _prefetch():
                    pltpu.sync_copy(pk_hbm.at[base + i + 1], npkt)
                    pltpu.sync_copy(wc_hbm.at[pl.ds((base + i + 1) * CH, CH)], nwv)
                    pltpu.async_copy(emb_hbm.at[npkt.at[1]], ndat, nsem)

                pltpu.make_async_copy(emb_hbm.at[pkt.at[1]], dat, sem).wait()

                def scale_group(g2, c2, dat=dat, wv_=wv_):
                    w16 = wv_[pl.ds(g2 * _LANES, _LANES)]
                    for el in range(_LANES):
                        wv = jnp.full((_LANES,), w16[el], dtype=jnp.float32)
                        e = g2 * _LANES + el
                        for j in range(D // _LANES):
                            sl = pl.ds(j * _LANES, _LANES)
                            dat[e, sl] = dat[e, sl] * wv
                    return c2

                lax.fori_loop(0, CH // _LANES, scale_group, 0)
                pltpu.sync_copy(dat, acc_sh.at[pkt.at[0]], add=True)
            return c

        lax.fori_loop(0, n_chunks // 2, pair, 0)
        plsc.subcore_barrier()
        pltpu.sync_copy(acc_sh.at[pl.ds(sid * rps, rps)],
                        out_hbm.at[pl.ds(cid * N + sid * rps, rps)])

    return k(packed, wchunk, embeds, zeros_slab)


def _pe_combine(p0, p1, simy, pos, W, b, n_user_blocks):
    N, D = p0.shape
    BM = 512

    def body(p0_ref, p1_ref, sim_ref, pos_ref, w_ref, b_ref, out_ref):
        i = pl.program_id(0)
        cur = p0_ref[...] + p1_ref[...]
        cur = jnp.where(i >= n_user_blocks, cur + sim_ref[...], cur)
        gate = jax.nn.sigmoid(
            jnp.dot(cur, w_ref[...], preferred_element_type=jnp.float32)
            + b_ref[...])
        out_ref[...] = cur + gate * pos_ref[...]

    return pl.pallas_call(
        body,
        grid=(N // BM,),
        in_specs=[
            pl.BlockSpec((BM, D), lambda i: (i, 0)),
            pl.BlockSpec((BM, D), lambda i: (i, 0)),
            pl.BlockSpec((BM, D), lambda i: (jnp.maximum(i - n_user_blocks, 0), 0)),
            pl.BlockSpec((BM, D), lambda i: (i, 0)),
            pl.BlockSpec((D, D), lambda i: (0, 0)),
            pl.BlockSpec((1, D), lambda i: (0, 0)),
        ],
        out_specs=pl.BlockSpec((BM, D), lambda i: (i, 0)),
        out_shape=jax.ShapeDtypeStruct((N, D), jnp.float32),
    )(p0, p1, simy, pos, W, b.reshape(1, D))


def _transformer(x, p, H):
    L, D = x.shape
    F = p['W1'].shape[1]
    dh = D // H
    BQ = 512
    scale = 1.0 / (dh ** 0.5)

    def ln(v, g, b):
        m = jnp.mean(v, axis=-1, keepdims=True)
        var = jnp.mean((v - m) * (v - m), axis=-1, keepdims=True)
        return (v - m) / jnp.sqrt(var + 1e-5) * g + b

    def body(x_ref, xb_ref, wq_ref, bq_ref, wk_ref, bk_ref, wv_ref, bv_ref,
             wo_ref, bo_ref, g1_ref, be1_ref, g2_ref, be2_ref,
             w1_ref, b1_ref, w2_ref, b2_ref, out_ref):
        xf = x_ref[...]
        xb = xb_ref[...]
        q = jnp.dot(xb, wq_ref[...], preferred_element_type=jnp.float32) + bq_ref[...]
        kk = jnp.dot(xf, wk_ref[...], preferred_element_type=jnp.float32) + bk_ref[...]
        vv = jnp.dot(xf, wv_ref[...], preferred_element_type=jnp.float32) + bv_ref[...]
        outs = []
        for h in range(H):
            sl = slice(h * dh, (h + 1) * dh)
            qh = q[:, sl]
            kh = kk[:, sl]
            vh = vv[:, sl]
            s = lax.dot_general(qh, kh, (((1,), (1,)), ((), ())),
                                preferred_element_type=jnp.float32) * scale
            m = jnp.max(s, axis=-1, keepdims=True)
            e = jnp.exp(s - m)
            denom = jnp.sum(e, axis=-1, keepdims=True)
            outs.append(
                jnp.dot(e, vh, preferred_element_type=jnp.float32) / denom)
        o = jnp.concatenate(outs, axis=-1)
        h1 = xb + jnp.dot(o, wo_ref[...], preferred_element_type=jnp.float32) + bo_ref[...]
        h1 = ln(h1, g1_ref[...], be1_ref[...])
        ff = jnp.maximum(
            jnp.dot(h1, w1_ref[...], preferred_element_type=jnp.float32) + b1_ref[...],
            0.0)
        ff = jnp.dot(ff, w2_ref[...], preferred_element_type=jnp.float32) + b2_ref[...]
        out_ref[...] = ln(h1 + ff, g2_ref[...], be2_ref[...])

    full = lambda shape: pl.BlockSpec(shape, lambda i: (0, 0))
    return pl.pallas_call(
        body,
        grid=(L // BQ,),
        in_specs=[
            pl.BlockSpec((L, D), lambda i: (0, 0)),
            pl.BlockSpec((BQ, D), lambda i: (i, 0)),
            full((D, D)), full((1, D)),
            full((D, D)), full((1, D)),
            full((D, D)), full((1, D)),
            full((D, D)), full((1, D)),
            full((1, D)), full((1, D)),
            full((1, D)), full((1, D)),
            full((D, F)), full((1, F)),
            full((F, D)), full((1, D)),
        ],
        out_specs=pl.BlockSpec((BQ, D), lambda i: (i, 0)),
        out_shape=jax.ShapeDtypeStruct((L, D), jnp.float32),
    )(x, x,
      p['Wq'], p['bq'].reshape(1, D),
      p['Wk'], p['bk'].reshape(1, D),
      p['Wv'], p['bv'].reshape(1, D),
      p['Wo'], p['bo'].reshape(1, D),
      p['ln1_g'].reshape(1, D), p['ln1_b'].reshape(1, D),
      p['ln2_g'].reshape(1, D), p['ln2_b'].reshape(1, D),
      p['W1'], p['b1'].reshape(1, F),
      p['W2'], p['b2'].reshape(1, D))


def _wsum(e0, e1, e2):
    N, D = e0.shape
    BM = 512

    def body(a_ref, b_ref, c_ref, out_ref):
        out_ref[...] = a_ref[...] + 0.75 * b_ref[...] + 0.5 * c_ref[...]

    spec = pl.BlockSpec((BM, D), lambda i: (i, 0))
    return pl.pallas_call(
        body,
        grid=(N // BM,),
        in_specs=[spec, spec, spec],
        out_specs=spec,
        out_shape=jax.ShapeDtypeStruct((N, D), jnp.float32),
    )(e0, e1, e2)


def kernel(edge_index, edge_weight, params):
    p = params
    U = p['user_emb'].shape[0]
    N, RAW = p['raw_emb'].shape
    D = p['W_proj'].shape[1]
    H = 2

    E = edge_weight.shape[0]
    total_chunks = E // _CH
    packed = jnp.stack([
        edge_index[0].reshape(total_chunks, _CH),
        edge_index[1].reshape(total_chunks, _CH),
    ], axis=1)
    wflat = edge_weight.astype(jnp.float32)
    alpha11 = jnp.reshape(p['alpha'], (1, 1)).astype(jnp.float32)
    learned = jnp.concatenate([p['user_emb'], p['item_emb']], axis=0)
    zeros_slab = jnp.zeros((N // _NS, D), jnp.float32)

    combined = _rawproj_combine(p['raw_emb'], p['W_proj'], p['b_proj'],
                                learned, alpha11)
    dinv = _dinv_kernel(p['visual_sim'])

    n_user_blocks = U // 512
    cur = combined
    stages = [combined]
    for _ in range(2):
        simy = _simprop(p['visual_sim'], dinv, cur[U:], alpha11)
        parts = _sc_scatter(packed, wflat, cur, zeros_slab)
        cur2 = _pe_combine(parts[:N], parts[N:], simy, p['pos_table'],
                           p['pe_gate_W'], p['pe_gate_b'], n_user_blocks)
        u = _transformer(cur2[:U], p['user_enc'], H)
        it = _transformer(cur2[U:], p['item_enc'], H)
        cur = jnp.concatenate([u, it], axis=0)
        stages.append(cur)

    final = _wsum(stages[0], stages[1], stages[2])
    return final, final[:U], final[U:]


# speedup vs baseline: 5.4942x; 1.0004x over previous
import functools

import jax
import jax.numpy as jnp
from jax import lax
from jax.experimental import pallas as pl
from jax.experimental.pallas import tpu as pltpu
from jax.experimental.pallas import tpu_sc as plsc

_NC = 2
_NS = 16
_LANES = 16


def _rawproj_combine(raw, Wp, bp, learned, alpha11):
    N, RAW = raw.shape
    D = Wp.shape[1]
    BM = 512

    def body(a_ref, raw_ref, w_ref, b_ref, learned_ref, out_ref):
        a = a_ref[0, 0]
        acc = jnp.dot(raw_ref[...], w_ref[...], preferred_element_type=jnp.float32)
        out_ref[...] = a * (acc + b_ref[...]) + (1.0 - a) * learned_ref[...]

    return pl.pallas_call(
        body,
        grid=(N // BM,),
        in_specs=[
            pl.BlockSpec(memory_space=pltpu.SMEM),
            pl.BlockSpec((BM, RAW), lambda i: (i, 0)),
            pl.BlockSpec((RAW, D), lambda i: (0, 0)),
            pl.BlockSpec((1, D), lambda i: (0, 0)),
            pl.BlockSpec((BM, D), lambda i: (i, 0)),
        ],
        out_specs=pl.BlockSpec((BM, D), lambda i: (i, 0)),
        out_shape=jax.ShapeDtypeStruct((N, D), jnp.float32),
    )(alpha11, raw, Wp, bp.reshape(1, D), learned)


def _dinv_kernel(S):
    I = S.shape[0]
    BM = 512

    def body(s_ref, out_ref):
        out_ref[...] = lax.rsqrt(jnp.sum(s_ref[...], axis=1, keepdims=True) + 1e-7)

    return pl.pallas_call(
        body,
        grid=(I // BM,),
        in_specs=[pl.BlockSpec((BM, I), lambda i: (i, 0))],
        out_specs=pl.BlockSpec((BM, 1), lambda i: (i, 0)),
        out_shape=jax.ShapeDtypeStruct((I, 1), jnp.float32),
    )(S)


def _simprop(S, dinv, x, alpha11):
    I = S.shape[0]
    D = x.shape[1]
    BM = 512
    BK = 512
    nk = I // BK

    def body(a_ref, s_ref, x_ref, dk_ref, dm_ref, out_ref):
        k = pl.program_id(1)

        @pl.when(k == 0)
        def _init():
            out_ref[...] = jnp.zeros_like(out_ref)

        xs = dk_ref[...] * x_ref[...]
        out_ref[...] += jnp.dot(s_ref[...], xs, preferred_element_type=jnp.float32)

        @pl.when(k == nk - 1)
        def _fini():
            out_ref[...] = a_ref[0, 0] * dm_ref[...] * out_ref[...]

    return pl.pallas_call(
        body,
        grid=(I // BM, nk),
        in_specs=[
            pl.BlockSpec(memory_space=pltpu.SMEM),
            pl.BlockSpec((BM, BK), lambda i, k: (i, k)),
            pl.BlockSpec((BK, D), lambda i, k: (k, 0)),
            pl.BlockSpec((BK, 1), lambda i, k: (k, 0)),
            pl.BlockSpec((BM, 1), lambda i, k: (i, 0)),
        ],
        out_specs=pl.BlockSpec((BM, D), lambda i, k: (i, 0)),
        out_shape=jax.ShapeDtypeStruct((I, D), jnp.float32),
        compiler_params=pltpu.CompilerParams(
            dimension_semantics=("parallel", "arbitrary")),
    )(alpha11, S, x, dinv, dinv)


_CH = 128


def _sc_scatter(packed, wchunk, embeds, zeros_slab):
    total_chunks, _, CH = packed.shape
    N, D = embeds.shape
    NW = _NC * _NS
    n_chunks = total_chunks // NW
    rps = N // _NS

    mesh = plsc.VectorSubcoreMesh(core_axis_name="c", subcore_axis_name="s")

    NBUF = 4

    @functools.partial(
        pl.kernel,
        mesh=mesh,
        compiler_params=pltpu.CompilerParams(use_tc_tiling_on_sc=False),
        out_type=jax.ShapeDtypeStruct((_NC * N, D), jnp.float32),
        scratch_types=(
            [pltpu.VMEM((2, CH), jnp.int32) for _ in range(NBUF)]
            + [pltpu.VMEM((CH,), jnp.float32) for _ in range(NBUF)]
            + [pltpu.VMEM((CH, D), jnp.float32) for _ in range(NBUF)]
            + [pltpu.VMEM_SHARED((N, D), jnp.float32)]
            + [pltpu.SemaphoreType.DMA for _ in range(2 * NBUF)]
        ),
    )
    def k(pk_hbm, wc_hbm, emb_hbm, zero_hbm, out_hbm, *refs):
        pkts = refs[0:NBUF]
        wvs = refs[NBUF:2 * NBUF]
        dats = refs[2 * NBUF:3 * NBUF]
        acc_sh = refs[3 * NBUF]
        gsems = refs[3 * NBUF + 1:4 * NBUF + 1]
        ssems = refs[4 * NBUF + 1:5 * NBUF + 1]
        cid = lax.axis_index("c")
        sid = lax.axis_index("s")
        wid = sid * _NC + cid
        base = wid * n_chunks

        pltpu.sync_copy(zero_hbm, acc_sh.at[pl.ds(sid * rps, rps)])
        pltpu.sync_copy(pk_hbm.at[base], pkts[0])
        pltpu.sync_copy(wc_hbm.at[pl.ds(base * CH, CH)], wvs[0])
        pltpu.async_copy(emb_hbm.at[pkts[0].at[1]], dats[0], gsems[0])
        plsc.subcore_barrier()

        def quad(q, c):
            for b in range(NBUF):
                i = NBUF * q + b
                pkt, wv_, dat, gsem, ssem = pkts[b], wvs[b], dats[b], gsems[b], ssems[b]
                nb = (b + 1) % NBUF
                npkt, nwv, ndat, ngsem, nssem = (
                    pkts[nb], wvs[nb], dats[nb], gsems[nb], ssems[nb])

                @pl.when(i + 1 < n_chunks)
                def _prefetch():
                    pltpu.sync_copy(pk_hbm.at[base + i + 1], npkt)
                    pltpu.sync_copy(
                        wc_hbm.at[pl.ds((base + i + 1) * CH, CH)], nwv)
                    pltpu.async_copy(emb_hbm.at[npkt.at[1]], ndat, ngsem)

                pltpu.make_async_copy(emb_hbm.at[pkt.at[1]], dat, gsem).wait()

                def _scale(g2, c2, dat=dat, wv_=wv_):
                    w16 = wv_[pl.ds(g2 * _LANES, _LANES)]
                    for el in range(_LANES):
                        wsplat = jnp.full((_LANES,), w16[el], dtype=jnp.float32)
                        e = g2 * _LANES + el
                        for j in range(D // _LANES):
                            sl = pl.ds(j * _LANES, _LANES)
                            dat[e, sl] = dat[e, sl] * wsplat
                    return c2

                lax.fori_loop(0, CH // _LANES, _scale, 0)

                pltpu.sync_copy(dat, acc_sh.at[pkt.at[0]], add=True)
            return c

        lax.fori_loop(0, n_chunks // NBUF, quad, 0)
        plsc.subcore_barrier()
        pltpu.sync_copy(acc_sh.at[pl.ds(sid * rps, rps)],
                        out_hbm.at[pl.ds(cid * N + sid * rps, rps)])

    return k(packed, wchunk, embeds, zeros_slab)


def _pe_combine(p0, p1, simy, pos, W, b, n_user_blocks):
    N, D = p0.shape
    BM = 512

    def body(p0_ref, p1_ref, sim_ref, pos_ref, w_ref, b_ref, out_ref):
        i = pl.program_id(0)
        cur = p0_ref[...] + p1_ref[...]
        cur = jnp.where(i >= n_user_blocks, cur + sim_ref[...], cur)
        gate = jax.nn.sigmoid(
            jnp.dot(cur, w_ref[...], preferred_element_type=jnp.float32)
            + b_ref[...])
        out_ref[...] = cur + gate * pos_ref[...]

    return pl.pallas_call(
        body,
        grid=(N // BM,),
        in_specs=[
            pl.BlockSpec((BM, D), lambda i: (i, 0)),
            pl.BlockSpec((BM, D), lambda i: (i, 0)),
            pl.BlockSpec((BM, D), lambda i: (jnp.maximum(i - n_user_blocks, 0), 0)),
            pl.BlockSpec((BM, D), lambda i: (i, 0)),
            pl.BlockSpec((D, D), lambda i: (0, 0)),
            pl.BlockSpec((1, D), lambda i: (0, 0)),
        ],
        out_specs=pl.BlockSpec((BM, D), lambda i: (i, 0)),
        out_shape=jax.ShapeDtypeStruct((N, D), jnp.float32),
    )(p0, p1, simy, pos, W, b.reshape(1, D))


def _transformer(x, p, H):
    L, D = x.shape
    F = p['W1'].shape[1]
    dh = D // H
    BQ = 512
    scale = 1.0 / (dh ** 0.5)

    def ln(v, g, b):
        m = jnp.mean(v, axis=-1, keepdims=True)
        var = jnp.mean((v - m) * (v - m), axis=-1, keepdims=True)
        return (v - m) / jnp.sqrt(var + 1e-5) * g + b

    def body(x_ref, xb_ref, wq_ref, bq_ref, wk_ref, bk_ref, wv_ref, bv_ref,
             wo_ref, bo_ref, g1_ref, be1_ref, g2_ref, be2_ref,
             w1_ref, b1_ref, w2_ref, b2_ref, out_ref):
        xf = x_ref[...]
        xb = xb_ref[...]
        q = jnp.dot(xb, wq_ref[...], preferred_element_type=jnp.float32) + bq_ref[...]
        kk = jnp.dot(xf, wk_ref[...], preferred_element_type=jnp.float32) + bk_ref[...]
        vv = jnp.dot(xf, wv_ref[...], preferred_element_type=jnp.float32) + bv_ref[...]
        outs = []
        for h in range(H):
            sl = slice(h * dh, (h + 1) * dh)
            qh = q[:, sl]
            kh = kk[:, sl]
            vh = vv[:, sl]
            s = lax.dot_general(qh, kh, (((1,), (1,)), ((), ())),
                                preferred_element_type=jnp.float32) * scale
            m = jnp.max(s, axis=-1, keepdims=True)
            e = jnp.exp(s - m)
            denom = jnp.sum(e, axis=-1, keepdims=True)
            outs.append(
                jnp.dot(e, vh, preferred_element_type=jnp.float32) / denom)
        o = jnp.concatenate(outs, axis=-1)
        h1 = xb + jnp.dot(o, wo_ref[...], preferred_element_type=jnp.float32) + bo_ref[...]
        h1 = ln(h1, g1_ref[...], be1_ref[...])
        ff = jnp.maximum(
            jnp.dot(h1, w1_ref[...], preferred_element_type=jnp.float32) + b1_ref[...],
            0.0)
        ff = jnp.dot(ff, w2_ref[...], preferred_element_type=jnp.float32) + b2_ref[...]
        out_ref[...] = ln(h1 + ff, g2_ref[...], be2_ref[...])

    full = lambda shape: pl.BlockSpec(shape, lambda i: (0, 0))
    return pl.pallas_call(
        body,
        grid=(L // BQ,),
        in_specs=[
            pl.BlockSpec((L, D), lambda i: (0, 0)),
            pl.BlockSpec((BQ, D), lambda i: (i, 0)),
            full((D, D)), full((1, D)),
            full((D, D)), full((1, D)),
            full((D, D)), full((1, D)),
            full((D, D)), full((1, D)),
            full((1, D)), full((1, D)),
            full((1, D)), full((1, D)),
            full((D, F)), full((1, F)),
            full((F, D)), full((1, D)),
        ],
        out_specs=pl.BlockSpec((BQ, D), lambda i: (i, 0)),
        out_shape=jax.ShapeDtypeStruct((L, D), jnp.float32),
    )(x, x,
      p['Wq'], p['bq'].reshape(1, D),
      p['Wk'], p['bk'].reshape(1, D),
      p['Wv'], p['bv'].reshape(1, D),
      p['Wo'], p['bo'].reshape(1, D),
      p['ln1_g'].reshape(1, D), p['ln1_b'].reshape(1, D),
      p['ln2_g'].reshape(1, D), p['ln2_b'].reshape(1, D),
      p['W1'], p['b1'].reshape(1, F),
      p['W2'], p['b2'].reshape(1, D))


def _wsum(e0, e1, e2):
    N, D = e0.shape
    BM = 512

    def body(a_ref, b_ref, c_ref, out_ref):
        out_ref[...] = a_ref[...] + 0.75 * b_ref[...] + 0.5 * c_ref[...]

    spec = pl.BlockSpec((BM, D), lambda i: (i, 0))
    return pl.pallas_call(
        body,
        grid=(N // BM,),
        in_specs=[spec, spec, spec],
        out_specs=spec,
        out_shape=jax.ShapeDtypeStruct((N, D), jnp.float32),
    )(e0, e1, e2)


def kernel(edge_index, edge_weight, params):
    p = params
    U = p['user_emb'].shape[0]
    N, RAW = p['raw_emb'].shape
    D = p['W_proj'].shape[1]
    H = 2

    E = edge_weight.shape[0]
    total_chunks = E // _CH
    packed = jnp.stack([
        edge_index[0].reshape(total_chunks, _CH),
        edge_index[1].reshape(total_chunks, _CH),
    ], axis=1)
    wflat = edge_weight.astype(jnp.float32)
    alpha11 = jnp.reshape(p['alpha'], (1, 1)).astype(jnp.float32)
    learned = jnp.concatenate([p['user_emb'], p['item_emb']], axis=0)
    zeros_slab = jnp.zeros((N // _NS, D), jnp.float32)

    combined = _rawproj_combine(p['raw_emb'], p['W_proj'], p['b_proj'],
                                learned, alpha11)
    dinv = _dinv_kernel(p['visual_sim'])

    n_user_blocks = U // 512
    cur = combined
    stages = [combined]
    for _ in range(2):
        simy = _simprop(p['visual_sim'], dinv, cur[U:], alpha11)
        parts = _sc_scatter(packed, wflat, cur, zeros_slab)
        cur2 = _pe_combine(parts[:N], parts[N:], simy, p['pos_table'],
                           p['pe_gate_W'], p['pe_gate_b'], n_user_blocks)
        u = _transformer(cur2[:U], p['user_enc'], H)
        it = _transformer(cur2[U:], p['item_enc'], H)
        cur = jnp.concatenate([u, it], axis=0)
        stages.append(cur)

    final = _wsum(stages[0], stages[1], stages[2])
    return final, final[:U], final[U:]


# speedup vs baseline: 5.7512x; 1.0468x over previous
import functools

import jax
import jax.numpy as jnp
from jax import lax
from jax.experimental import pallas as pl
from jax.experimental.pallas import tpu as pltpu
from jax.experimental.pallas import tpu_sc as plsc

_NC = 2
_NS = 16
_LANES = 16


def _rawproj_combine(raw, Wp, bp, learned, alpha11):
    N, RAW = raw.shape
    D = Wp.shape[1]
    BM = 512

    def body(a_ref, raw_ref, w_ref, b_ref, learned_ref, out_ref):
        a = a_ref[0, 0]
        acc = jnp.dot(raw_ref[...], w_ref[...], preferred_element_type=jnp.float32)
        out_ref[...] = a * (acc + b_ref[...]) + (1.0 - a) * learned_ref[...]

    return pl.pallas_call(
        body,
        grid=(N // BM,),
        in_specs=[
            pl.BlockSpec(memory_space=pltpu.SMEM),
            pl.BlockSpec((BM, RAW), lambda i: (i, 0)),
            pl.BlockSpec((RAW, D), lambda i: (0, 0)),
            pl.BlockSpec((1, D), lambda i: (0, 0)),
            pl.BlockSpec((BM, D), lambda i: (i, 0)),
        ],
        out_specs=pl.BlockSpec((BM, D), lambda i: (i, 0)),
        out_shape=jax.ShapeDtypeStruct((N, D), jnp.float32),
    )(alpha11, raw, Wp, bp.reshape(1, D), learned)


def _dinv_kernel(S):
    I = S.shape[0]
    BM = 512

    def body(s_ref, out_ref):
        out_ref[...] = lax.rsqrt(jnp.sum(s_ref[...], axis=1, keepdims=True) + 1e-7)

    return pl.pallas_call(
        body,
        grid=(I // BM,),
        in_specs=[pl.BlockSpec((BM, I), lambda i: (i, 0))],
        out_specs=pl.BlockSpec((BM, 1), lambda i: (i, 0)),
        out_shape=jax.ShapeDtypeStruct((I, 1), jnp.float32),
    )(S)


def _simprop(S, dinv, x, alpha11):
    I = S.shape[0]
    D = x.shape[1]
    BM = 512
    BK = 512
    nk = I // BK

    def body(a_ref, s_ref, x_ref, dk_ref, dm_ref, out_ref):
        k = pl.program_id(1)

        @pl.when(k == 0)
        def _init():
            out_ref[...] = jnp.zeros_like(out_ref)

        xs = dk_ref[...] * x_ref[...]
        out_ref[...] += jnp.dot(s_ref[...], xs, preferred_element_type=jnp.float32)

        @pl.when(k == nk - 1)
        def _fini():
            out_ref[...] = a_ref[0, 0] * dm_ref[...] * out_ref[...]

    return pl.pallas_call(
        body,
        grid=(I // BM, nk),
        in_specs=[
            pl.BlockSpec(memory_space=pltpu.SMEM),
            pl.BlockSpec((BM, BK), lambda i, k: (i, k)),
            pl.BlockSpec((BK, D), lambda i, k: (k, 0)),
            pl.BlockSpec((BK, 1), lambda i, k: (k, 0)),
            pl.BlockSpec((BM, 1), lambda i, k: (i, 0)),
        ],
        out_specs=pl.BlockSpec((BM, D), lambda i, k: (i, 0)),
        out_shape=jax.ShapeDtypeStruct((I, D), jnp.float32),
        compiler_params=pltpu.CompilerParams(
            dimension_semantics=("parallel", "arbitrary")),
    )(alpha11, S, x, dinv, dinv)


_CH = 128


def _sc_scatter(packed, wchunk, embeds, zeros_slab):
    total_chunks, _, CH = packed.shape
    N, D = embeds.shape
    NW = _NC * _NS
    n_chunks = total_chunks // NW
    rps = N // _NS

    mesh = plsc.VectorSubcoreMesh(core_axis_name="c", subcore_axis_name="s")

    NBUF = 4

    @functools.partial(
        pl.kernel,
        mesh=mesh,
        compiler_params=pltpu.CompilerParams(use_tc_tiling_on_sc=False),
        out_type=jax.ShapeDtypeStruct((_NC * N, D), jnp.float32),
        scratch_types=(
            [pltpu.VMEM((2, CH), jnp.int32) for _ in range(NBUF)]
            + [pltpu.VMEM((CH,), jnp.float32) for _ in range(NBUF)]
            + [pltpu.VMEM((CH, D), jnp.float32) for _ in range(NBUF)]
            + [pltpu.VMEM_SHARED((N, D), jnp.float32)]
            + [pltpu.SemaphoreType.DMA for _ in range(2 * NBUF)]
        ),
    )
    def k(pk_hbm, wc_hbm, emb_hbm, zero_hbm, out_hbm, *refs):
        pkts = refs[0:NBUF]
        wvs = refs[NBUF:2 * NBUF]
        dats = refs[2 * NBUF:3 * NBUF]
        acc_sh = refs[3 * NBUF]
        gsems = refs[3 * NBUF + 1:4 * NBUF + 1]
        ssems = refs[4 * NBUF + 1:5 * NBUF + 1]
        cid = lax.axis_index("c")
        sid = lax.axis_index("s")
        wid = sid * _NC + cid
        base = wid * n_chunks

        pltpu.sync_copy(zero_hbm, acc_sh.at[pl.ds(sid * rps, rps)])
        pltpu.sync_copy(pk_hbm.at[base], pkts[0])
        pltpu.sync_copy(wc_hbm.at[pl.ds(base * CH, CH)], wvs[0])
        pltpu.async_copy(emb_hbm.at[pkts[0].at[1]], dats[0], gsems[0])
        plsc.subcore_barrier()

        def quad(q, c):
            for b in range(NBUF):
                i = NBUF * q + b
                pkt, wv_, dat, gsem, ssem = pkts[b], wvs[b], dats[b], gsems[b], ssems[b]
                nb = (b + 1) % NBUF
                npkt, nwv, ndat, ngsem, nssem = (
                    pkts[nb], wvs[nb], dats[nb], gsems[nb], ssems[nb])

                @pl.when(i >= NBUF - 1)
                def _drain():
                    pltpu.make_async_copy(
                        ndat, acc_sh.at[npkt.at[0]], nssem).wait()

                @pl.when(i + 1 < n_chunks)
                def _prefetch():
                    pltpu.sync_copy(pk_hbm.at[base + i + 1], npkt)
                    pltpu.sync_copy(
                        wc_hbm.at[pl.ds((base + i + 1) * CH, CH)], nwv)
                    pltpu.async_copy(emb_hbm.at[npkt.at[1]], ndat, ngsem)

                pltpu.make_async_copy(emb_hbm.at[pkt.at[1]], dat, gsem).wait()

                def _scale(g2, c2, dat=dat, wv_=wv_):
                    w16 = wv_[pl.ds(g2 * _LANES, _LANES)]
                    for el in range(_LANES):
                        wsplat = jnp.full((_LANES,), w16[el], dtype=jnp.float32)
                        e = g2 * _LANES + el
                        for j in range(D // _LANES):
                            sl = pl.ds(j * _LANES, _LANES)
                            dat[e, sl] = dat[e, sl] * wsplat
                    return c2

                lax.fori_loop(0, CH // _LANES, _scale, 0)

                pltpu.async_copy(dat, acc_sh.at[pkt.at[0]], ssem, add=True)
            return c

        lax.fori_loop(0, n_chunks // NBUF, quad, 0)
        for b in range(1, NBUF):
            pltpu.make_async_copy(
                dats[b], acc_sh.at[pkts[b].at[0]], ssems[b]).wait()
        plsc.subcore_barrier()
        pltpu.sync_copy(acc_sh.at[pl.ds(sid * rps, rps)],
                        out_hbm.at[pl.ds(cid * N + sid * rps, rps)])

    return k(packed, wchunk, embeds, zeros_slab)


def _pe_combine(p0, p1, simy, pos, W, b, n_user_blocks):
    N, D = p0.shape
    BM = 512

    def body(p0_ref, p1_ref, sim_ref, pos_ref, w_ref, b_ref, out_ref):
        i = pl.program_id(0)
        cur = p0_ref[...] + p1_ref[...]
        cur = jnp.where(i >= n_user_blocks, cur + sim_ref[...], cur)
        gate = jax.nn.sigmoid(
            jnp.dot(cur, w_ref[...], preferred_element_type=jnp.float32)
            + b_ref[...])
        out_ref[...] = cur + gate * pos_ref[...]

    return pl.pallas_call(
        body,
        grid=(N // BM,),
        in_specs=[
            pl.BlockSpec((BM, D), lambda i: (i, 0)),
            pl.BlockSpec((BM, D), lambda i: (i, 0)),
            pl.BlockSpec((BM, D), lambda i: (jnp.maximum(i - n_user_blocks, 0), 0)),
            pl.BlockSpec((BM, D), lambda i: (i, 0)),
            pl.BlockSpec((D, D), lambda i: (0, 0)),
            pl.BlockSpec((1, D), lambda i: (0, 0)),
        ],
        out_specs=pl.BlockSpec((BM, D), lambda i: (i, 0)),
        out_shape=jax.ShapeDtypeStruct((N, D), jnp.float32),
    )(p0, p1, simy, pos, W, b.reshape(1, D))


def _transformer(x, p, H):
    L, D = x.shape
    F = p['W1'].shape[1]
    dh = D // H
    BQ = 512
    scale = 1.0 / (dh ** 0.5)

    def ln(v, g, b):
        m = jnp.mean(v, axis=-1, keepdims=True)
        var = jnp.mean((v - m) * (v - m), axis=-1, keepdims=True)
        return (v - m) / jnp.sqrt(var + 1e-5) * g + b

    def body(x_ref, xb_ref, wq_ref, bq_ref, wk_ref, bk_ref, wv_ref, bv_ref,
             wo_ref, bo_ref, g1_ref, be1_ref, g2_ref, be2_ref,
             w1_ref, b1_ref, w2_ref, b2_ref, out_ref):
        xf = x_ref[...]
        xb = xb_ref[...]
        q = jnp.dot(xb, wq_ref[...], preferred_element_type=jnp.float32) + bq_ref[...]
        kk = jnp.dot(xf, wk_ref[...], preferred_element_type=jnp.float32) + bk_ref[...]
        vv = jnp.dot(xf, wv_ref[...], preferred_element_type=jnp.float32) + bv_ref[...]
        outs = []
        for h in range(H):
            sl = slice(h * dh, (h + 1) * dh)
            qh = q[:, sl]
            kh = kk[:, sl]
            vh = vv[:, sl]
            s = lax.dot_general(qh, kh, (((1,), (1,)), ((), ())),
                                preferred_element_type=jnp.float32) * scale
            m = jnp.max(s, axis=-1, keepdims=True)
            e = jnp.exp(s - m)
            denom = jnp.sum(e, axis=-1, keepdims=True)
            outs.append(
                jnp.dot(e, vh, preferred_element_type=jnp.float32) / denom)
        o = jnp.concatenate(outs, axis=-1)
        h1 = xb + jnp.dot(o, wo_ref[...], preferred_element_type=jnp.float32) + bo_ref[...]
        h1 = ln(h1, g1_ref[...], be1_ref[...])
        ff = jnp.maximum(
            jnp.dot(h1, w1_ref[...], preferred_element_type=jnp.float32) + b1_ref[...],
            0.0)
        ff = jnp.dot(ff, w2_ref[...], preferred_element_type=jnp.float32) + b2_ref[...]
        out_ref[...] = ln(h1 + ff, g2_ref[...], be2_ref[...])

    full = lambda shape: pl.BlockSpec(shape, lambda i: (0, 0))
    return pl.pallas_call(
        body,
        grid=(L // BQ,),
        in_specs=[
            pl.BlockSpec((L, D), lambda i: (0, 0)),
            pl.BlockSpec((BQ, D), lambda i: (i, 0)),
            full((D, D)), full((1, D)),
            full((D, D)), full((1, D)),
            full((D, D)), full((1, D)),
            full((D, D)), full((1, D)),
            full((1, D)), full((1, D)),
            full((1, D)), full((1, D)),
            full((D, F)), full((1, F)),
            full((F, D)), full((1, D)),
        ],
        out_specs=pl.BlockSpec((BQ, D), lambda i: (i, 0)),
        out_shape=jax.ShapeDtypeStruct((L, D), jnp.float32),
    )(x, x,
      p['Wq'], p['bq'].reshape(1, D),
      p['Wk'], p['bk'].reshape(1, D),
      p['Wv'], p['bv'].reshape(1, D),
      p['Wo'], p['bo'].reshape(1, D),
      p['ln1_g'].reshape(1, D), p['ln1_b'].reshape(1, D),
      p['ln2_g'].reshape(1, D), p['ln2_b'].reshape(1, D),
      p['W1'], p['b1'].reshape(1, F),
      p['W2'], p['b2'].reshape(1, D))


def _wsum(e0, e1, e2):
    N, D = e0.shape
    BM = 512

    def body(a_ref, b_ref, c_ref, out_ref):
        out_ref[...] = a_ref[...] + 0.75 * b_ref[...] + 0.5 * c_ref[...]

    spec = pl.BlockSpec((BM, D), lambda i: (i, 0))
    return pl.pallas_call(
        body,
        grid=(N // BM,),
        in_specs=[spec, spec, spec],
        out_specs=spec,
        out_shape=jax.ShapeDtypeStruct((N, D), jnp.float32),
    )(e0, e1, e2)


def kernel(edge_index, edge_weight, params):
    p = params
    U = p['user_emb'].shape[0]
    N, RAW = p['raw_emb'].shape
    D = p['W_proj'].shape[1]
    H = 2

    E = edge_weight.shape[0]
    total_chunks = E // _CH
    packed = jnp.stack([
        edge_index[0].reshape(total_chunks, _CH),
        edge_index[1].reshape(total_chunks, _CH),
    ], axis=1)
    wflat = edge_weight.astype(jnp.float32)
    alpha11 = jnp.reshape(p['alpha'], (1, 1)).astype(jnp.float32)
    learned = jnp.concatenate([p['user_emb'], p['item_emb']], axis=0)
    zeros_slab = jnp.zeros((N // _NS, D), jnp.float32)

    combined = _rawproj_combine(p['raw_emb'], p['W_proj'], p['b_proj'],
                                learned, alpha11)
    dinv = _dinv_kernel(p['visual_sim'])

    n_user_blocks = U // 512
    cur = combined
    stages = [combined]
    for _ in range(2):
        simy = _simprop(p['visual_sim'], dinv, cur[U:], alpha11)
        parts = _sc_scatter(packed, wflat, cur, zeros_slab)
        cur2 = _pe_combine(parts[:N], parts[N:], simy, p['pos_table'],
                           p['pe_gate_W'], p['pe_gate_b'], n_user_blocks)
        u = _transformer(cur2[:U], p['user_enc'], H)
        it = _transformer(cur2[U:], p['item_enc'], H)
        cur = jnp.concatenate([u, it], axis=0)
        stages.append(cur)

    final = _wsum(stages[0], stages[1], stages[2])
    return final, final[:U], final[U:]


# speedup vs baseline: 6.0572x; 1.0532x over previous
import functools

import jax
import jax.numpy as jnp
from jax import lax
from jax.experimental import pallas as pl
from jax.experimental.pallas import tpu as pltpu
from jax.experimental.pallas import tpu_sc as plsc

_NC = 2
_NS = 16
_LANES = 16


def _rawproj_combine(raw, Wp, bp, learned, alpha11):
    N, RAW = raw.shape
    D = Wp.shape[1]
    BM = 512

    def body(a_ref, raw_ref, w_ref, b_ref, learned_ref, out_ref):
        a = a_ref[0, 0]
        acc = jnp.dot(raw_ref[...], w_ref[...], preferred_element_type=jnp.float32)
        out_ref[...] = a * (acc + b_ref[...]) + (1.0 - a) * learned_ref[...]

    return pl.pallas_call(
        body,
        grid=(N // BM,),
        in_specs=[
            pl.BlockSpec(memory_space=pltpu.SMEM),
            pl.BlockSpec((BM, RAW), lambda i: (i, 0)),
            pl.BlockSpec((RAW, D), lambda i: (0, 0)),
            pl.BlockSpec((1, D), lambda i: (0, 0)),
            pl.BlockSpec((BM, D), lambda i: (i, 0)),
        ],
        out_specs=pl.BlockSpec((BM, D), lambda i: (i, 0)),
        out_shape=jax.ShapeDtypeStruct((N, D), jnp.float32),
    )(alpha11, raw, Wp, bp.reshape(1, D), learned)


def _dinv_kernel(S):
    I = S.shape[0]
    BM = 512

    def body(s_ref, out_ref):
        out_ref[...] = lax.rsqrt(jnp.sum(s_ref[...], axis=1, keepdims=True) + 1e-7)

    return pl.pallas_call(
        body,
        grid=(I // BM,),
        in_specs=[pl.BlockSpec((BM, I), lambda i: (i, 0))],
        out_specs=pl.BlockSpec((BM, 1), lambda i: (i, 0)),
        out_shape=jax.ShapeDtypeStruct((I, 1), jnp.float32),
    )(S)


def _simprop(S, dinv, x, alpha11):
    I = S.shape[0]
    D = x.shape[1]
    BM = 512
    BK = 512
    nk = I // BK

    def body(a_ref, s_ref, x_ref, dk_ref, dm_ref, out_ref):
        k = pl.program_id(1)

        @pl.when(k == 0)
        def _init():
            out_ref[...] = jnp.zeros_like(out_ref)

        xs = dk_ref[...] * x_ref[...]
        out_ref[...] += jnp.dot(s_ref[...], xs, preferred_element_type=jnp.float32)

        @pl.when(k == nk - 1)
        def _fini():
            out_ref[...] = a_ref[0, 0] * dm_ref[...] * out_ref[...]

    return pl.pallas_call(
        body,
        grid=(I // BM, nk),
        in_specs=[
            pl.BlockSpec(memory_space=pltpu.SMEM),
            pl.BlockSpec((BM, BK), lambda i, k: (i, k)),
            pl.BlockSpec((BK, D), lambda i, k: (k, 0)),
            pl.BlockSpec((BK, 1), lambda i, k: (k, 0)),
            pl.BlockSpec((BM, 1), lambda i, k: (i, 0)),
        ],
        out_specs=pl.BlockSpec((BM, D), lambda i, k: (i, 0)),
        out_shape=jax.ShapeDtypeStruct((I, D), jnp.float32),
        compiler_params=pltpu.CompilerParams(
            dimension_semantics=("parallel", "arbitrary")),
    )(alpha11, S, x, dinv, dinv)


_CH = 128


def _sc_scatter(packed, wchunk, embeds, zeros_slab):
    total_chunks, _, CH = packed.shape
    N, D = embeds.shape
    NW = _NC * _NS
    n_chunks = total_chunks // NW
    rps = N // _NS

    mesh = plsc.VectorSubcoreMesh(core_axis_name="c", subcore_axis_name="s")

    NBUF = 4

    @functools.partial(
        pl.kernel,
        mesh=mesh,
        compiler_params=pltpu.CompilerParams(use_tc_tiling_on_sc=False),
        out_type=jax.ShapeDtypeStruct((_NC * N, D), jnp.float32),
        scratch_types=(
            [pltpu.VMEM((2, CH), jnp.int32) for _ in range(NBUF)]
            + [pltpu.VMEM((CH,), jnp.float32) for _ in range(NBUF)]
            + [pltpu.VMEM((CH, D), jnp.float32) for _ in range(NBUF)]
            + [pltpu.VMEM_SHARED((N, D), jnp.float32)]
            + [pltpu.SemaphoreType.DMA for _ in range(2 * NBUF)]
        ),
    )
    def k(pk_hbm, wc_hbm, emb_hbm, zero_hbm, out_hbm, *refs):
        pkts = refs[0:NBUF]
        wvs = refs[NBUF:2 * NBUF]
        dats = refs[2 * NBUF:3 * NBUF]
        acc_sh = refs[3 * NBUF]
        gsems = refs[3 * NBUF + 1:4 * NBUF + 1]
        ssems = refs[4 * NBUF + 1:5 * NBUF + 1]
        cid = lax.axis_index("c")
        sid = lax.axis_index("s")
        wid = sid * _NC + cid
        base = wid * n_chunks

        pltpu.sync_copy(zero_hbm, acc_sh.at[pl.ds(sid * rps, rps)])
        pltpu.sync_copy(pk_hbm.at[base], pkts[0])
        pltpu.sync_copy(wc_hbm.at[pl.ds(base * CH, CH)], wvs[0])
        pltpu.async_copy(emb_hbm.at[pkts[0].at[1]], dats[0], gsems[0])
        plsc.subcore_barrier()

        def quad(q, c):
            for b in range(NBUF):
                i = NBUF * q + b
                pkt, wv_, dat, gsem, ssem = pkts[b], wvs[b], dats[b], gsems[b], ssems[b]
                nb = (b + 1) % NBUF
                npkt, nwv, ndat, ngsem, nssem = (
                    pkts[nb], wvs[nb], dats[nb], gsems[nb], ssems[nb])

                @pl.when(i >= NBUF - 1)
                def _drain():
                    pltpu.make_async_copy(
                        ndat, acc_sh.at[npkt.at[0]], nssem).wait()

                @pl.when(i + 1 < n_chunks)
                def _prefetch():
                    pltpu.sync_copy(pk_hbm.at[base + i + 1], npkt)
                    pltpu.sync_copy(
                        wc_hbm.at[pl.ds((base + i + 1) * CH, CH)], nwv)
                    pltpu.async_copy(emb_hbm.at[npkt.at[1]], ndat, ngsem)

                pltpu.make_async_copy(emb_hbm.at[pkt.at[1]], dat, gsem).wait()

                for g2 in range(CH // _LANES):
                    w16 = wv_[pl.ds(g2 * _LANES, _LANES)]
                    for el in range(_LANES):
                        wsplat = jnp.full((_LANES,), w16[el], dtype=jnp.float32)
                        e = g2 * _LANES + el
                        for j in range(D // _LANES):
                            sl = pl.ds(j * _LANES, _LANES)
                            dat[e, sl] = dat[e, sl] * wsplat

                pltpu.async_copy(dat, acc_sh.at[pkt.at[0]], ssem, add=True)
            return c

        lax.fori_loop(0, n_chunks // NBUF, quad, 0)
        for b in range(1, NBUF):
            pltpu.make_async_copy(
                dats[b], acc_sh.at[pkts[b].at[0]], ssems[b]).wait()
        plsc.subcore_barrier()
        pltpu.sync_copy(acc_sh.at[pl.ds(sid * rps, rps)],
                        out_hbm.at[pl.ds(cid * N + sid * rps, rps)])

    return k(packed, wchunk, embeds, zeros_slab)


def _pe_combine(p0, p1, simy, pos, W, b, n_user_blocks):
    N, D = p0.shape
    BM = 512

    def body(p0_ref, p1_ref, sim_ref, pos_ref, w_ref, b_ref, out_ref):
        i = pl.program_id(0)
        cur = p0_ref[...] + p1_ref[...]
        cur = jnp.where(i >= n_user_blocks, cur + sim_ref[...], cur)
        gate = jax.nn.sigmoid(
            jnp.dot(cur, w_ref[...], preferred_element_type=jnp.float32)
            + b_ref[...])
        out_ref[...] = cur + gate * pos_ref[...]

    return pl.pallas_call(
        body,
        grid=(N // BM,),
        in_specs=[
            pl.BlockSpec((BM, D), lambda i: (i, 0)),
            pl.BlockSpec((BM, D), lambda i: (i, 0)),
            pl.BlockSpec((BM, D), lambda i: (jnp.maximum(i - n_user_blocks, 0), 0)),
            pl.BlockSpec((BM, D), lambda i: (i, 0)),
            pl.BlockSpec((D, D), lambda i: (0, 0)),
            pl.BlockSpec((1, D), lambda i: (0, 0)),
        ],
        out_specs=pl.BlockSpec((BM, D), lambda i: (i, 0)),
        out_shape=jax.ShapeDtypeStruct((N, D), jnp.float32),
    )(p0, p1, simy, pos, W, b.reshape(1, D))


def _transformer_pair(x2, ps, H):
    N2, D = x2.shape
    L = N2 // 2
    F = ps['W1'].shape[2]
    dh = D // H
    BQ = 1024
    G = L // BQ
    scale = 1.0 / (dh ** 0.5)

    def ln(v, g, b):
        m = jnp.mean(v, axis=-1, keepdims=True)
        var = jnp.mean((v - m) * (v - m), axis=-1, keepdims=True)
        return (v - m) / jnp.sqrt(var + 1e-5) * g + b

    def body(x_ref, xb_ref, wq_ref, bq_ref, wk_ref, bk_ref, wv_ref, bv_ref,
             wo_ref, bo_ref, g1_ref, be1_ref, g2_ref, be2_ref,
             w1_ref, b1_ref, w2_ref, b2_ref, out_ref, kk_s, vv_s):
        i = pl.program_id(0)

        @pl.when(i % G == 0)
        def _kv():
            xf = x_ref[...]
            kk_s[...] = (jnp.dot(xf, wk_ref[0],
                                 preferred_element_type=jnp.float32)
                         + bk_ref[0])
            vv_s[...] = (jnp.dot(xf, wv_ref[0],
                                 preferred_element_type=jnp.float32)
                         + bv_ref[0])

        xb = xb_ref[...]
        q = (jnp.dot(xb, wq_ref[0], preferred_element_type=jnp.float32)
             + bq_ref[0])
        kk = kk_s[...]
        vv = vv_s[...]
        outs = []
        for h in range(H):
            sl = slice(h * dh, (h + 1) * dh)
            qh = q[:, sl]
            kh = kk[:, sl]
            vh = vv[:, sl]
            s = lax.dot_general(qh, kh, (((1,), (1,)), ((), ())),
                                preferred_element_type=jnp.float32) * scale
            m = jnp.max(s, axis=-1, keepdims=True)
            e = jnp.exp(s - m)
            denom = jnp.sum(e, axis=-1, keepdims=True)
            outs.append(
                jnp.dot(e, vh, preferred_element_type=jnp.float32) / denom)
        o = jnp.concatenate(outs, axis=-1)
        h1 = (xb + jnp.dot(o, wo_ref[0], preferred_element_type=jnp.float32)
              + bo_ref[0])
        h1 = ln(h1, g1_ref[0], be1_ref[0])
        ff = jnp.maximum(
            jnp.dot(h1, w1_ref[0], preferred_element_type=jnp.float32)
            + b1_ref[0], 0.0)
        ff = jnp.dot(ff, w2_ref[0], preferred_element_type=jnp.float32) + b2_ref[0]
        out_ref[...] = ln(h1 + ff, g2_ref[0], be2_ref[0])

    half = lambda i: (i // G, 0)
    mat = lambda s2: pl.BlockSpec((1,) + s2, lambda i: (i // G, 0, 0))
    vec = lambda n: pl.BlockSpec((1, 1, n), lambda i: (i // G, 0, 0))
    return pl.pallas_call(
        body,
        grid=(2 * G,),
        in_specs=[
            pl.BlockSpec((L, D), half),
            pl.BlockSpec((BQ, D), lambda i: (i, 0)),
            mat((D, D)), vec(D),
            mat((D, D)), vec(D),
            mat((D, D)), vec(D),
            mat((D, D)), vec(D),
            vec(D), vec(D),
            vec(D), vec(D),
            mat((D, F)), vec(F),
            mat((F, D)), vec(D),
        ],
        out_specs=pl.BlockSpec((BQ, D), lambda i: (i, 0)),
        out_shape=jax.ShapeDtypeStruct((N2, D), jnp.float32),
        scratch_shapes=[
            pltpu.VMEM((L, D), jnp.float32),
            pltpu.VMEM((L, D), jnp.float32),
        ],
        compiler_params=pltpu.CompilerParams(
            dimension_semantics=("arbitrary",)),
    )(x2, x2,
      ps['Wq'], ps['bq'], ps['Wk'], ps['bk'],
      ps['Wv'], ps['bv'], ps['Wo'], ps['bo'],
      ps['ln1_g'], ps['ln1_b'], ps['ln2_g'], ps['ln2_b'],
      ps['W1'], ps['b1'], ps['W2'], ps['b2'])


def _wsum(e0, e1, e2):
    N, D = e0.shape
    BM = 512

    def body(a_ref, b_ref, c_ref, out_ref):
        out_ref[...] = a_ref[...] + 0.75 * b_ref[...] + 0.5 * c_ref[...]

    spec = pl.BlockSpec((BM, D), lambda i: (i, 0))
    return pl.pallas_call(
        body,
        grid=(N // BM,),
        in_specs=[spec, spec, spec],
        out_specs=spec,
        out_shape=jax.ShapeDtypeStruct((N, D), jnp.float32),
    )(e0, e1, e2)


def kernel(edge_index, edge_weight, params):
    p = params
    U = p['user_emb'].shape[0]
    N, RAW = p['raw_emb'].shape
    D = p['W_proj'].shape[1]
    H = 2

    E = edge_weight.shape[0]
    total_chunks = E // _CH
    packed = jnp.stack([
        edge_index[0].reshape(total_chunks, _CH),
        edge_index[1].reshape(total_chunks, _CH),
    ], axis=1)
    wflat = edge_weight.astype(jnp.float32)
    alpha11 = jnp.reshape(p['alpha'], (1, 1)).astype(jnp.float32)
    learned = jnp.concatenate([p['user_emb'], p['item_emb']], axis=0)
    zeros_slab = jnp.zeros((N // _NS, D), jnp.float32)

    combined = _rawproj_combine(p['raw_emb'], p['W_proj'], p['b_proj'],
                                learned, alpha11)
    ue, ie = p['user_enc'], p['item_enc']
    enc = {}
    for kname in ue:
        st = jnp.stack([ue[kname], ie[kname]])
        enc[kname] = st.reshape(2, 1, -1) if st.ndim == 2 else st
    dinv = _dinv_kernel(p['visual_sim'])

    n_user_blocks = U // 512
    cur = combined
    stages = [combined]
    for _ in range(2):
        simy = _simprop(p['visual_sim'], dinv, cur[U:], alpha11)
        parts = _sc_scatter(packed, wflat, cur, zeros_slab)
        cur2 = _pe_combine(parts[:N], parts[N:], simy, p['pos_table'],
                           p['pe_gate_W'], p['pe_gate_b'], n_user_blocks)
        cur = _transformer_pair(cur2, enc, H)
        stages.append(cur)

    final = _wsum(stages[0], stages[1], stages[2])
    return final, final[:U], final[U:]
